# baseline (speedup 1.0000x reference)
import functools

import jax
import jax.numpy as jnp
from jax import lax
from jax.experimental import pallas as pl
from jax.experimental.pallas import tpu as pltpu
from jax.experimental.pallas import tpu_sc as plsc

_EMBED_DIM = 64
_BATCH = 16384

_NC = 2
_NS = 16
_NW = _NC * _NS
_B_PER_W = _BATCH // _NW
_CHUNK = 128
_NCHUNK = _B_PER_W // _CHUNK

_mesh = plsc.VectorSubcoreMesh(core_axis_name="c", subcore_axis_name="s")


@functools.partial(
    pl.kernel,
    mesh=_mesh,
    out_type=jax.ShapeDtypeStruct((_BATCH, _EMBED_DIM), jnp.float32),
    scratch_types=[
        pltpu.VMEM((_B_PER_W,), jnp.int32),
        pltpu.VMEM((_B_PER_W, _EMBED_DIM), jnp.float32),
        pltpu.SemaphoreType.DMA,
    ],
    compiler_params=pltpu.CompilerParams(use_tc_tiling_on_sc=False),
)
def _gather_rows(table_hbm, idx_hbm, out_hbm, idx_v, rows_v, sem):
    wid = lax.axis_index("s") * _NC + lax.axis_index("c")
    base = wid * _B_PER_W
    pltpu.sync_copy(idx_hbm.at[pl.ds(base, _B_PER_W)], idx_v)
    copies = [
        pltpu.async_copy(
            table_hbm.at[idx_v.at[pl.ds(j * _CHUNK, _CHUNK)]],
            rows_v.at[pl.ds(j * _CHUNK, _CHUNK)],
            sem,
        )
        for j in range(_NCHUNK)
    ]
    for c in copies:
        c.wait()
    pltpu.sync_copy(rows_v, out_hbm.at[pl.ds(base, _B_PER_W)])


def kernel(user_indices, weight):
    return _gather_rows(weight, user_indices.astype(jnp.int32))


# speedup vs baseline: 3.0235x; 3.0235x over previous
import functools

import jax
import jax.numpy as jnp
from jax import lax
from jax.experimental import pallas as pl
from jax.experimental.pallas import tpu as pltpu
from jax.experimental.pallas import tpu_sc as plsc

_EMBED_DIM = 64
_BATCH = 16384
_TBL_COLS = 1000000

_NC = 2
_NS = 16
_NW = _NC * _NS
_B_PER_W = _BATCH // _NW
_LANES = 16
_RING = 8
_CHUNK = 128

_mesh = plsc.VectorSubcoreMesh(core_axis_name="c", subcore_axis_name="s")


@functools.partial(
    pl.kernel,
    mesh=_mesh,
    out_type=jax.ShapeDtypeStruct((_EMBED_DIM, _BATCH), jnp.float32),
    scratch_types=[
        pltpu.VMEM((_B_PER_W + _LANES,), jnp.int32),
        pltpu.VMEM((_RING, _EMBED_DIM, 128), jnp.float32),
        pltpu.VMEM((_EMBED_DIM, _CHUNK), jnp.float32),
        pltpu.SemaphoreType.DMA((_RING,)),
    ],
    compiler_params=pltpu.CompilerParams(needs_layout_passes=False),
)
def _gather_rows(tt_hbm, idx_hbm, out_hbm, idx_v, g_v, o_v, sem):
    wid = lax.axis_index("s") * _NC + lax.axis_index("c")
    base = wid * _B_PER_W
    pltpu.sync_copy(idx_hbm.at[pl.ds(base, _B_PER_W)], idx_v.at[pl.ds(0, _B_PER_W)])
    lanes = lax.iota(jnp.int32, _LANES)

    def read_idx(i):
        return idx_v[pl.ds(i, _LANES)][0]

    def fetch(i, slot):
        off = pl.multiple_of((read_idx(i) >> 7) * 128, 128)
        pltpu.async_copy(tt_hbm.at[:, pl.ds(off, 128)], g_v.at[slot], sem.at[slot])

    for s in range(_RING):
        fetch(s, s)

    for c in range(_B_PER_W // _CHUNK):

        def group_body(k, carry, c=c):
            i0 = c * _CHUNK + k * _RING
            for s in range(_RING):
                i = i0 + s
                pltpu.make_async_copy(
                    tt_hbm.at[:, pl.ds(0, 128)], g_v.at[s], sem.at[s]
                ).wait()
                col = jnp.full((_LANES,), read_idx(i) & 127, jnp.int32)
                j = jnp.full((_LANES,), k * _RING + s, jnp.int32)
                for m in range(_EMBED_DIM // _LANES):
                    d = m * _LANES + lanes
                    vals = plsc.load_gather(g_v.at[s], [d, col])
                    plsc.store_scatter(o_v, [d, j], vals)
                @pl.when(i + _RING < _B_PER_W)
                def _():
                    fetch(i + _RING, s)
            return carry

        lax.fori_loop(0, _CHUNK // _RING, group_body, 0)
        pltpu.sync_copy(
            o_v, out_hbm.at[:, pl.ds(base + c * _CHUNK, _CHUNK)]
        )


def kernel(user_indices, weight):
    out_t = _gather_rows(weight.T, user_indices.astype(jnp.int32))
    return out_t.T
